# baseline (speedup 1.0000x reference)
import functools

import jax
import jax.numpy as jnp
from jax import lax
from jax.experimental import pallas as pl
from jax.experimental.pallas import tpu as pltpu
from jax.experimental.pallas import tpu_sc as plsc

_CHUNK = 128


@functools.lru_cache(maxsize=None)
def _make_gather(B, D):
    info = plsc.get_sparse_core_info()
    NC, NS = info.num_cores, info.num_subcores
    NW = NC * NS
    b_per_w = B // NW
    n_chunks = b_per_w // _CHUNK
    mesh = plsc.VectorSubcoreMesh(core_axis_name="c", subcore_axis_name="s")

    @functools.partial(
        pl.kernel,
        mesh=mesh,
        compiler_params=pltpu.CompilerParams(use_tc_tiling_on_sc=False),
        out_type=[
            jax.ShapeDtypeStruct((B, D), jnp.float32),
            jax.ShapeDtypeStruct((B, D), jnp.float32),
        ],
        scratch_types=[
            pltpu.VMEM((n_chunks, _CHUNK), jnp.int32),
            pltpu.VMEM((n_chunks, _CHUNK), jnp.int32),
            pltpu.VMEM((_CHUNK, D), jnp.float32),
            pltpu.VMEM((_CHUNK, D), jnp.float32),
            pltpu.SemaphoreType.DMA,
            pltpu.SemaphoreType.DMA,
        ],
    )
    def k(iw_hbm, ow_hbm, itab_hbm, otab_hbm, out_i, out_o,
          iw_v, ow_v, rows0, rows1, sem0, sem1):
        wid = lax.axis_index("s") * NC + lax.axis_index("c")
        pltpu.sync_copy(iw_hbm.at[pl.ds(wid * n_chunks, n_chunks)], iw_v)
        pltpu.sync_copy(ow_hbm.at[pl.ds(wid * n_chunks, n_chunks)], ow_v)

        tasks = ([(itab_hbm, iw_v, out_i, c) for c in range(n_chunks)]
                 + [(otab_hbm, ow_v, out_o, c) for c in range(n_chunks)])
        bufs = (rows0, rows1)
        sems = (sem0, sem1)
        copies = [None, None]

        def fire(t):
            tab, idx, _, c = tasks[t]
            copies[t % 2] = pltpu.async_copy(
                tab.at[idx.at[c]], bufs[t % 2], sems[t % 2])

        fire(0)
        if len(tasks) > 1:
            fire(1)
        for t in range(len(tasks)):
            copies[t % 2].wait()
            _, _, out, c = tasks[t]
            base = wid * b_per_w + c * _CHUNK
            pltpu.sync_copy(bufs[t % 2], out.at[pl.ds(base, _CHUNK)])
            if t + 2 < len(tasks):
                fire(t + 2)

    return k


def kernel(input_words, output_words, input_table, output_table):
    B = input_words.shape[0]
    D = input_table.shape[1]
    iw = input_words.astype(jnp.int32).reshape(B // _CHUNK, _CHUNK)
    ow = output_words.astype(jnp.int32).reshape(B // _CHUNK, _CHUNK)
    out_i, out_o = _make_gather(B, D)(iw, ow, input_table, output_table)
    return (out_i, out_o)
